# baseline (speedup 1.0000x reference)
import functools

import jax
import jax.numpy as jnp
from jax import lax
from jax.experimental import pallas as pl
from jax.experimental.pallas import tpu as pltpu
from jax.experimental.pallas import tpu_sc as plsc

VOCAB = 1000000
EMB = 64
B = 4096
L = 200

NC = 2
NS = 16
NW = NC * NS
RPW = B // NW
H0 = 104
H1 = L - H0
NBUF = 4

NTC = VOCAB // 128
COLS_PER_W = NTC // NW + 1

_mesh_a = plsc.VectorSubcoreMesh(
    core_axis_name="c", subcore_axis_name="s", num_cores=NC, num_subcores=NS
)
_mesh_b = plsc.VectorSubcoreMesh(
    core_axis_name="c", subcore_axis_name="s", num_cores=NC, num_subcores=NS
)


@functools.partial(
    pl.kernel,
    out_type=jax.ShapeDtypeStruct((VOCAB // 2, 128), jnp.float32),
    mesh=_mesh_a,
    scratch_types=[
        pltpu.VMEM((4, EMB, 128), jnp.float32),
        pltpu.VMEM((4, EMB, 128), jnp.float32),
        [pltpu.SemaphoreType.DMA] * 4,
        [pltpu.SemaphoreType.DMA] * 4,
    ],
    compiler_params=pltpu.CompilerParams(use_tc_tiling_on_sc=True,
                                         needs_layout_passes=False),
)
def _detile(tt_hbm, out_hbm, x_v, y_v, sin, sout):
    wid = lax.axis_index("s") * NC + lax.axis_index("c")
    lane = lax.iota(jnp.int32, 16)
    rl = lax.shift_right_logical(lane, 1)
    cl = (lane & 1) * EMB
    rl_t = [rl + 8 * t for t in range(8)]

    def col(g):
        return wid + NW * g

    def valid(g):
        return col(g) <= NTC - 1

    def fire_in(g, p):
        off = pl.multiple_of(128 * col(g), 128)
        pltpu.async_copy(tt_hbm.at[:, pl.ds(off, 128)], x_v.at[p], sin[p])

    def wait_in(p):
        pltpu.make_async_copy(tt_hbm.at[:, pl.ds(0, 128)], x_v.at[p],
                              sin[p]).wait()

    def fire_out(g, p):
        off = pl.multiple_of(64 * col(g), 64)
        pltpu.async_copy(y_v.at[p], out_hbm.at[pl.ds(off, 64)], sout[p])

    def wait_out(p):
        pltpu.make_async_copy(y_v.at[0], out_hbm.at[pl.ds(0, 64)],
                              sout[p]).wait()

    def transpose(p):
        @plsc.parallel_loop(0, EMB, unroll=4)
        def e_body(e):
            ce = cl + e
            for t in range(8):
                v = x_v[p, e, pl.ds(16 * t, 16)]
                plsc.store_scatter(y_v.at[p], [rl_t[t], ce], v)

    for b in range(3):
        fire_in(b, b)

    def group_body(q, carry):
        for b in range(4):
            g = 4 * q + b

            @pl.when(valid(g + 3))
            def _():
                fire_in(g + 3, (b + 3) % 4)

            wait_in(b)

            @pl.when(q > 0)
            def _():
                wait_out(b)

            transpose(b)
            fire_out(g, b)
        return carry

    lax.fori_loop(0, (COLS_PER_W - 1) // 4, group_body, 0)

    g_last = COLS_PER_W - 1

    @pl.when(valid(g_last))
    def _():
        wait_in(0)
        wait_out(0)
        transpose(0)
        fire_out(g_last, 0)

    for b in range(4):
        wait_out(b)


@functools.partial(
    pl.kernel,
    out_type=jax.ShapeDtypeStruct((B, EMB), jnp.float32),
    mesh=_mesh_b,
    scratch_types=[
        pltpu.VMEM((RPW, H0), jnp.int32),
        pltpu.VMEM((RPW, H1), jnp.int32),
        pltpu.VMEM((NBUF, L, EMB), jnp.float32),
        pltpu.VMEM((RPW, EMB), jnp.float32),
        [pltpu.SemaphoreType.DMA] * NBUF,
    ],
    compiler_params=pltpu.CompilerParams(use_tc_tiling_on_sc=False),
)
def _encode(ids_hbm, table_hbm, out_hbm, ids0_v, ids1_v, emb_v, out_v, sems):
    wid = lax.axis_index("s") * NC + lax.axis_index("c")
    base = wid * RPW

    pltpu.sync_copy(ids_hbm.at[pl.ds(base, RPW), pl.ds(0, H0)], ids0_v)
    pltpu.sync_copy(ids_hbm.at[pl.ds(base, RPW), pl.ds(H0, H1)], ids1_v)

    inv_l = jnp.full((16,), 1.0 / L, dtype=jnp.float32)

    def fire(r, b):
        pltpu.async_copy(table_hbm.at[ids0_v.at[r]],
                         emb_v.at[b, pl.ds(0, H0)], sems[b])
        pltpu.async_copy(table_hbm.at[ids1_v.at[r]],
                         emb_v.at[b, pl.ds(H0, H1)], sems[b])

    def drain(b):
        pltpu.make_async_copy(table_hbm.at[ids0_v.at[0]],
                              emb_v.at[b, pl.ds(0, H0)], sems[b]).wait()
        pltpu.make_async_copy(table_hbm.at[ids1_v.at[0]],
                              emb_v.at[b, pl.ds(H0, H1)], sems[b]).wait()

    def accumulate(r, b):
        def acc_body(i, acc):
            a0, a1, a2, a3 = acc
            l = 4 * i
            for u in range(4):
                a0 = a0 + emb_v[b, l + u, pl.ds(0, 16)]
                a1 = a1 + emb_v[b, l + u, pl.ds(16, 16)]
                a2 = a2 + emb_v[b, l + u, pl.ds(32, 16)]
                a3 = a3 + emb_v[b, l + u, pl.ds(48, 16)]
            return (a0, a1, a2, a3)

        z = jnp.zeros((16,), jnp.float32)
        a0, a1, a2, a3 = lax.fori_loop(0, L // 4, acc_body, (z, z, z, z))
        out_v[r, pl.ds(0, 16)] = a0 * inv_l
        out_v[r, pl.ds(16, 16)] = a1 * inv_l
        out_v[r, pl.ds(32, 16)] = a2 * inv_l
        out_v[r, pl.ds(48, 16)] = a3 * inv_l

    for b in range(NBUF):
        fire(b, b)

    def group_body(g, carry):
        r0 = NBUF * g
        for b in range(NBUF):
            r = r0 + b
            drain(b)
            accumulate(r, b)
            fire(r + NBUF, b)
        return carry

    lax.fori_loop(0, RPW // NBUF - 1, group_body, 0)

    r_last = RPW - NBUF
    for b in range(NBUF):
        drain(b)
        accumulate(r_last + b, b)

    pltpu.sync_copy(out_v, out_hbm.at[pl.ds(base, RPW)])


def kernel(token_ids, table):
    tlin2 = _detile(table.T)
    tail = table[NTC * 128:].reshape(32, 128)
    tlin2 = lax.dynamic_update_slice(tlin2, tail, (NTC * 64, 0))
    tlin = tlin2.reshape(VOCAB, EMB)
    return _encode(token_ids, tlin)


# speedup vs baseline: 3.3378x; 3.3378x over previous
import functools

import jax
import jax.numpy as jnp
from jax import lax
from jax.experimental import pallas as pl
from jax.experimental.pallas import tpu as pltpu
from jax.experimental.pallas import tpu_sc as plsc

VOCAB = 1000000
EMB = 64
B = 4096
L = 200

NC = 2
NS = 16
NW = NC * NS
RPW = B // NW
H0 = 104
H1 = L - H0
NBUF = 4

NTC = VOCAB // 128
COLS_PER_W = NTC // NW + 1

_mesh_a = plsc.VectorSubcoreMesh(
    core_axis_name="c", subcore_axis_name="s", num_cores=NC, num_subcores=NS
)
_mesh_b = plsc.VectorSubcoreMesh(
    core_axis_name="c", subcore_axis_name="s", num_cores=NC, num_subcores=NS
)


@functools.partial(
    pl.kernel,
    out_type=jax.ShapeDtypeStruct((VOCAB // 2, 128), jnp.float32),
    mesh=_mesh_a,
    scratch_types=[
        pltpu.VMEM((4, EMB, 128), jnp.float32),
        pltpu.VMEM((EMB * 129,), jnp.float32),
        pltpu.VMEM((EMB * 129,), jnp.float32),
        pltpu.VMEM((4, EMB, 128), jnp.float32),
        [pltpu.SemaphoreType.DMA] * 4,
        [pltpu.SemaphoreType.DMA] * 4,
    ],
    compiler_params=pltpu.CompilerParams(use_tc_tiling_on_sc=True,
                                         needs_layout_passes=False),
)
def _detile(tt_hbm, out_hbm, x_v, s0_v, s1_v, y_v, sin, sout):
    s_v = (s0_v, s1_v)
    wid = lax.axis_index("s") * NC + lax.axis_index("c")
    lane = lax.iota(jnp.int32, 16)
    rl = lax.shift_right_logical(lane, 1)
    cl = (lane & 1) * EMB
    b_t = [rl * 129 + cl + 1032 * t for t in range(8)]

    def col(g):
        return wid + NW * g

    def valid(g):
        return col(g) <= NTC - 1

    def fire_in(g, p):
        off = pl.multiple_of(128 * col(g), 128)
        pltpu.async_copy(tt_hbm.at[:, pl.ds(off, 128)], x_v.at[p], sin[p])

    def wait_in(p):
        pltpu.make_async_copy(tt_hbm.at[:, pl.ds(0, 128)], x_v.at[p],
                              sin[p]).wait()

    def fire_out(g, p):
        off = pl.multiple_of(64 * col(g), 64)
        pltpu.async_copy(y_v.at[p], out_hbm.at[pl.ds(off, 64)], sout[p])

    def wait_out(p):
        pltpu.make_async_copy(y_v.at[0], out_hbm.at[pl.ds(0, 64)],
                              sout[p]).wait()

    def transpose(p, sp, q):
        @plsc.parallel_loop(0, EMB, unroll=4)
        def e_body(e):
            for t in range(8):
                v = x_v[p, e, pl.ds(16 * t, 16)]
                plsc.store_scatter(s_v[sp], [b_t[t] + e], v)

        @plsc.parallel_loop(0, EMB, unroll=4)
        def r_body(r):
            for t in range(8):
                y_v[p, r, pl.ds(16 * t, 16)] = s_v[sp][pl.ds(129 * r + 16 * t, 16)]

    for b in range(3):
        fire_in(b, b)

    def group_body(q, carry):
        for b in range(4):
            g = 4 * q + b

            @pl.when(valid(g + 3))
            def _():
                fire_in(g + 3, (b + 3) % 4)

            wait_in(b)

            @pl.when(q > 0)
            def _():
                wait_out(b)

            transpose(b, b % 2, q)
            fire_out(g, b)
        return carry

    lax.fori_loop(0, (COLS_PER_W - 1) // 4, group_body, 0)

    g_last = COLS_PER_W - 1

    @pl.when(valid(g_last))
    def _():
        wait_in(0)
        wait_out(0)
        transpose(0, 0, 0)
        fire_out(g_last, 0)

    for b in range(4):
        wait_out(b)


@functools.partial(
    pl.kernel,
    out_type=jax.ShapeDtypeStruct((B, EMB), jnp.float32),
    mesh=_mesh_b,
    scratch_types=[
        pltpu.VMEM((RPW, H0), jnp.int32),
        pltpu.VMEM((RPW, H1), jnp.int32),
        pltpu.VMEM((NBUF, L, EMB), jnp.float32),
        pltpu.VMEM((RPW, EMB), jnp.float32),
        [pltpu.SemaphoreType.DMA] * NBUF,
    ],
    compiler_params=pltpu.CompilerParams(use_tc_tiling_on_sc=False),
)
def _encode(ids_hbm, table_hbm, out_hbm, ids0_v, ids1_v, emb_v, out_v, sems):
    wid = lax.axis_index("s") * NC + lax.axis_index("c")
    base = wid * RPW

    pltpu.sync_copy(ids_hbm.at[pl.ds(base, RPW), pl.ds(0, H0)], ids0_v)
    pltpu.sync_copy(ids_hbm.at[pl.ds(base, RPW), pl.ds(H0, H1)], ids1_v)

    inv_l = jnp.full((16,), 1.0 / L, dtype=jnp.float32)

    def fire(r, b):
        pltpu.async_copy(table_hbm.at[ids0_v.at[r]],
                         emb_v.at[b, pl.ds(0, H0)], sems[b])
        pltpu.async_copy(table_hbm.at[ids1_v.at[r]],
                         emb_v.at[b, pl.ds(H0, H1)], sems[b])

    def drain(b):
        pltpu.make_async_copy(table_hbm.at[ids0_v.at[0]],
                              emb_v.at[b, pl.ds(0, H0)], sems[b]).wait()
        pltpu.make_async_copy(table_hbm.at[ids1_v.at[0]],
                              emb_v.at[b, pl.ds(H0, H1)], sems[b]).wait()

    def accumulate(r, b):
        def acc_body(i, acc):
            a0, a1, a2, a3 = acc
            l = 4 * i
            for u in range(4):
                a0 = a0 + emb_v[b, l + u, pl.ds(0, 16)]
                a1 = a1 + emb_v[b, l + u, pl.ds(16, 16)]
                a2 = a2 + emb_v[b, l + u, pl.ds(32, 16)]
                a3 = a3 + emb_v[b, l + u, pl.ds(48, 16)]
            return (a0, a1, a2, a3)

        z = jnp.zeros((16,), jnp.float32)
        a0, a1, a2, a3 = lax.fori_loop(0, L // 4, acc_body, (z, z, z, z))
        out_v[r, pl.ds(0, 16)] = a0 * inv_l
        out_v[r, pl.ds(16, 16)] = a1 * inv_l
        out_v[r, pl.ds(32, 16)] = a2 * inv_l
        out_v[r, pl.ds(48, 16)] = a3 * inv_l

    for b in range(NBUF):
        fire(b, b)

    def group_body(g, carry):
        r0 = NBUF * g
        for b in range(NBUF):
            r = r0 + b
            drain(b)
            accumulate(r, b)
            fire(r + NBUF, b)
        return carry

    lax.fori_loop(0, RPW // NBUF - 1, group_body, 0)

    r_last = RPW - NBUF
    for b in range(NBUF):
        drain(b)
        accumulate(r_last + b, b)

    pltpu.sync_copy(out_v, out_hbm.at[pl.ds(base, RPW)])


def kernel(token_ids, table):
    tlin2 = _detile(table.T)
    tail = table[NTC * 128:].reshape(32, 128)
    tlin2 = lax.dynamic_update_slice(tlin2, tail, (NTC * 64, 0))
    tlin = tlin2.reshape(VOCAB, EMB)
    return _encode(token_ids, tlin)
